# baseline (speedup 1.0000x reference)
import functools

import jax
import jax.numpy as jnp
from jax import lax
from jax.experimental import pallas as pl
from jax.experimental.pallas import tpu as pltpu
from jax.experimental.pallas import tpu_sc as plsc

B = 16384
F = 26
PER_FIELD = 38461
D = 16
V = F * PER_FIELD

NUM_CORES = 2
FPC = F // NUM_CORES

W = 38656
ROWB = W + 64
A_LAST = 961280
TAIL0 = A_LAST + W
NTAIL = V - TAIL0
NTAIL_PAD = 64
TTAIL = TAIL0 - (F - 1) * PER_FIELD

UNROLL = 4

_mesh = plsc.VectorSubcoreMesh(core_axis_name="c", subcore_axis_name="s")


@functools.partial(
    pl.kernel,
    mesh=_mesh,
    out_type=jax.ShapeDtypeStruct((F, D, B), jnp.float32),
    scratch_types=[
        pltpu.VMEM((ROWB,), jnp.float32),
        pltpu.VMEM((ROWB,), jnp.float32),
        pltpu.VMEM((B,), jnp.int32),
        pltpu.VMEM((B,), jnp.int32),
        pltpu.VMEM((B,), jnp.float32),
        pltpu.VMEM((NTAIL_PAD,), jnp.float32),
        pltpu.SemaphoreType.DMA,
        pltpu.SemaphoreType.DMA,
        pltpu.SemaphoreType.DMA,
        pltpu.SemaphoreType.DMA,
        pltpu.SemaphoreType.DMA,
    ],
    compiler_params=pltpu.CompilerParams(
        use_tc_tiling_on_sc=True, needs_layout_passes=False
    ),
)
def _fm_kernel(
    tfT, tabT, tail_lin, outT,
    row0, row1, idx0, idx1, obuf, tailbuf,
    ssem0, ssem1, isem0, isem1, osem,
):
    cid = lax.axis_index("c")
    sid = lax.axis_index("s")

    rows = (row0, row1)
    idxs = (idx0, idx1)
    ssems = (ssem0, ssem1)
    isems = (isem0, isem1)

    pltpu.sync_copy(tail_lin.at[pl.ds(sid * NTAIL_PAD, NTAIL_PAD)], tailbuf)

    def slab_start(f):
        if f == F - 1:
            return A_LAST
        return (f * PER_FIELD // 128) * 128

    def fetch(fi, slot):
        for cb in range(NUM_CORES):
            fs = cb * FPC + fi
            a = slab_start(fs)

            @pl.when(cid == cb)
            def _():
                pltpu.async_copy(
                    tabT.at[sid, pl.ds(a, W)], rows[slot].at[pl.ds(0, W)],
                    ssems[slot],
                )
                pltpu.async_copy(tfT.at[fs, :], idxs[slot], isems[slot])

    def wait_fetch(slot):
        pltpu.make_async_copy(
            tabT.at[sid, pl.ds(0, W)], rows[slot].at[pl.ds(0, W)], ssems[slot]
        ).wait()
        pltpu.make_async_copy(tfT.at[0, :], idxs[slot], isems[slot]).wait()

    def put(fi):
        for cb in range(NUM_CORES):
            fs = cb * FPC + fi

            @pl.when(cid == cb)
            def _():
                pltpu.async_copy(obuf, outT.at[fs, sid, :], osem)

    def wait_put():
        pltpu.make_async_copy(obuf, outT.at[0, sid, :], osem).wait()

    fetch(0, 0)
    for fi in range(FPC):
        slot = fi % 2
        wait_fetch(slot)
        if fi + 1 < FPC:
            fetch(fi + 1, 1 - slot)

        rowbuf = rows[slot]
        idxbuf = idxs[slot]

        if fi > 0:
            wait_put()

        def body(jj, c):
            for u in range(UNROLL):
                j = jj * UNROLL + u
                idx16 = idxbuf[pl.ds(j * 16, 16)]
                obuf[pl.ds(j * 16, 16)] = plsc.load_gather(rowbuf, [idx16])
            return c

        lax.fori_loop(0, B // (16 * UNROLL), body, 0)

        if fi == FPC - 1:

            @pl.when(cid == NUM_CORES - 1)
            def _():
                def patch(j, c):
                    pos = j * 16 + lax.iota(jnp.int32, 16)
                    idx16 = idxbuf[pl.ds(j * 16, 16)]
                    tloc = jnp.maximum(idx16 - W, 0)
                    tval = plsc.load_gather(tailbuf, [tloc])
                    plsc.store_scatter(obuf, [pos], tval, mask=idx16 >= W)
                    return c

                lax.fori_loop(0, B // 16, patch, 0)

        put(fi)
    wait_put()


def _slab_starts():
    starts = [(f * PER_FIELD // 128) * 128 for f in range(F - 1)] + [A_LAST]
    return jnp.asarray(starts, dtype=jnp.int32)


def kernel(token_fields, table):
    tfT = token_fields.T
    tabT = table.T
    offs = jnp.arange(F, dtype=jnp.int32) * PER_FIELD - _slab_starts()
    local = tfT + offs[:, None]
    tail = jnp.pad(tabT[:, TAIL0:], ((0, 0), (0, NTAIL_PAD - NTAIL)))
    tail_lin = tail.reshape(D * NTAIL_PAD)
    outT = _fm_kernel(local, tabT, tail_lin)
    return jnp.transpose(outT, (2, 0, 1))


# speedup vs baseline: 1.1046x; 1.1046x over previous
import functools

import jax
import jax.numpy as jnp
from jax import lax
from jax.experimental import pallas as pl
from jax.experimental.pallas import tpu as pltpu
from jax.experimental.pallas import tpu_sc as plsc

B = 16384
F = 26
PER_FIELD = 38461
D = 16
V = F * PER_FIELD

NUM_CORES = 2
FPC = F // NUM_CORES

W = 38656
ROWB = W + 64
A_LAST = 961280
TAIL0 = A_LAST + W
NTAIL = V - TAIL0
NTAIL_PAD = 64
TTAIL = TAIL0 - (F - 1) * PER_FIELD

UNROLL = 4

_mesh = plsc.VectorSubcoreMesh(core_axis_name="c", subcore_axis_name="s")


@functools.partial(
    pl.kernel,
    mesh=_mesh,
    out_type=jax.ShapeDtypeStruct((F, D, B), jnp.float32),
    scratch_types=[
        pltpu.VMEM((ROWB,), jnp.float32),
        pltpu.VMEM((ROWB,), jnp.float32),
        pltpu.VMEM((B,), jnp.int32),
        pltpu.VMEM((B,), jnp.int32),
        pltpu.VMEM((B,), jnp.float32),
        pltpu.VMEM((NTAIL_PAD,), jnp.float32),
        pltpu.SemaphoreType.DMA,
        pltpu.SemaphoreType.DMA,
        pltpu.SemaphoreType.DMA,
        pltpu.SemaphoreType.DMA,
        pltpu.SemaphoreType.DMA,
    ],
    compiler_params=pltpu.CompilerParams(
        use_tc_tiling_on_sc=True, needs_layout_passes=False
    ),
)
def _fm_kernel(
    tfT, tabT, tail_lin, outT,
    row0, row1, idx0, idx1, obuf, tailbuf,
    ssem0, ssem1, isem0, isem1, osem,
):
    cid = lax.axis_index("c")
    sid = lax.axis_index("s")

    rows = (row0, row1)
    idxs = (idx0, idx1)
    ssems = (ssem0, ssem1)
    isems = (isem0, isem1)

    pltpu.sync_copy(tail_lin.at[pl.ds(sid * NTAIL_PAD, NTAIL_PAD)], tailbuf)

    def slab_start(f):
        if f == F - 1:
            return A_LAST
        return (f * PER_FIELD // 128) * 128

    def fetch(fi, slot):
        for cb in range(NUM_CORES):
            fs = cb * FPC + fi
            a = slab_start(fs)

            @pl.when(cid == cb)
            def _():
                pltpu.async_copy(
                    tabT.at[sid, pl.ds(a, W)], rows[slot].at[pl.ds(0, W)],
                    ssems[slot],
                )
                pltpu.async_copy(tfT.at[fs, :], idxs[slot], isems[slot])

    def wait_fetch(slot):
        pltpu.make_async_copy(
            tabT.at[sid, pl.ds(0, W)], rows[slot].at[pl.ds(0, W)], ssems[slot]
        ).wait()
        pltpu.make_async_copy(tfT.at[0, :], idxs[slot], isems[slot]).wait()

    def put(fi):
        for cb in range(NUM_CORES):
            fs = cb * FPC + fi

            @pl.when(cid == cb)
            def _():
                pltpu.async_copy(obuf, outT.at[fs, sid, :], osem)

    def wait_put():
        pltpu.make_async_copy(obuf, outT.at[0, sid, :], osem).wait()

    fetch(0, 0)
    for fi in range(FPC):
        slot = fi % 2
        wait_fetch(slot)
        if fi + 1 < FPC:
            fetch(fi + 1, 1 - slot)

        rowbuf = rows[slot]
        idxbuf = idxs[slot]

        if fi > 0:
            wait_put()

        def body(jj, c):
            for u in range(UNROLL):
                j = jj * UNROLL + u
                idx16 = idxbuf[pl.ds(j * 16, 16)]
                obuf[pl.ds(j * 16, 16)] = plsc.load_gather(rowbuf, [idx16])
            return c

        def body_tail(jj, c):
            for u in range(UNROLL):
                j = jj * UNROLL + u
                idx16 = idxbuf[pl.ds(j * 16, 16)]
                main = plsc.load_gather(rowbuf, [idx16])
                tval = plsc.load_gather(tailbuf, [jnp.maximum(idx16 - W, 0)])
                obuf[pl.ds(j * 16, 16)] = jnp.where(idx16 >= W, tval, main)
            return c

        if fi == FPC - 1:

            @pl.when(cid < NUM_CORES - 1)
            def _():
                lax.fori_loop(0, B // (16 * UNROLL), body, 0)

            @pl.when(cid == NUM_CORES - 1)
            def _():
                lax.fori_loop(0, B // (16 * UNROLL), body_tail, 0)

        else:
            lax.fori_loop(0, B // (16 * UNROLL), body, 0)

        put(fi)
    wait_put()


def _slab_starts():
    starts = [(f * PER_FIELD // 128) * 128 for f in range(F - 1)] + [A_LAST]
    return jnp.asarray(starts, dtype=jnp.int32)


def kernel(token_fields, table):
    tfT = token_fields.T
    tabT = table.T
    offs = jnp.arange(F, dtype=jnp.int32) * PER_FIELD - _slab_starts()
    local = tfT + offs[:, None]
    tail = jnp.pad(tabT[:, TAIL0:], ((0, 0), (0, NTAIL_PAD - NTAIL)))
    tail_lin = tail.reshape(D * NTAIL_PAD)
    outT = _fm_kernel(local, tabT, tail_lin)
    return jnp.transpose(outT, (2, 0, 1))


# speedup vs baseline: 1.3622x; 1.2332x over previous
import functools

import jax
import jax.numpy as jnp
from jax import lax
from jax.experimental import pallas as pl
from jax.experimental.pallas import tpu as pltpu
from jax.experimental.pallas import tpu_sc as plsc

B = 16384
F = 26
PER_FIELD = 38461
D = 16
V = F * PER_FIELD

NUM_CORES = 2
FPC = F // NUM_CORES

W = 38656
ROWB = W + 64
A_LAST = 961280
TAIL0 = A_LAST + W
NTAIL = V - TAIL0
NTAIL_PAD = 64
TTAIL = TAIL0 - (F - 1) * PER_FIELD

UNROLL = 4

_mesh = plsc.VectorSubcoreMesh(core_axis_name="c", subcore_axis_name="s")


@functools.partial(
    pl.kernel,
    mesh=_mesh,
    out_type=jax.ShapeDtypeStruct((F, D, B), jnp.float32),
    scratch_types=[
        pltpu.VMEM((ROWB,), jnp.float32),
        pltpu.VMEM((ROWB,), jnp.float32),
        pltpu.VMEM((B,), jnp.int32),
        pltpu.VMEM((B,), jnp.int32),
        pltpu.VMEM((B,), jnp.float32),
        pltpu.VMEM((NTAIL_PAD,), jnp.float32),
        pltpu.SemaphoreType.DMA,
        pltpu.SemaphoreType.DMA,
        pltpu.SemaphoreType.DMA,
        pltpu.SemaphoreType.DMA,
        pltpu.SemaphoreType.DMA,
    ],
    compiler_params=pltpu.CompilerParams(
        use_tc_tiling_on_sc=True, needs_layout_passes=False
    ),
)
def _fm_kernel(
    tfT, tabT, tail_lin, outT,
    row0, row1, idx0, idx1, obuf, tailbuf,
    ssem0, ssem1, isem0, isem1, osem,
):
    cid = lax.axis_index("c")
    sid = lax.axis_index("s")

    rows = (row0, row1)
    idxs = (idx0, idx1)
    ssems = (ssem0, ssem1)
    isems = (isem0, isem1)

    pltpu.sync_copy(tail_lin.at[pl.ds(sid * NTAIL_PAD, NTAIL_PAD)], tailbuf)

    def slab_start(f):
        if f == F - 1:
            return A_LAST
        return (f * PER_FIELD // 128) * 128

    def fetch(fi, slot):
        for cb in range(NUM_CORES):
            fs = cb * FPC + fi
            a = slab_start(fs)

            @pl.when(cid == cb)
            def _():
                pltpu.async_copy(
                    tabT.at[sid, pl.ds(a, W)], rows[slot].at[pl.ds(0, W)],
                    ssems[slot],
                )
                pltpu.async_copy(tfT.at[fs, :], idxs[slot], isems[slot])

    def wait_fetch(slot):
        pltpu.make_async_copy(
            tabT.at[sid, pl.ds(0, W)], rows[slot].at[pl.ds(0, W)], ssems[slot]
        ).wait()
        pltpu.make_async_copy(tfT.at[0, :], idxs[slot], isems[slot]).wait()

    def put(fi):
        for cb in range(NUM_CORES):
            fs = cb * FPC + fi

            @pl.when(cid == cb)
            def _():
                pltpu.async_copy(obuf, outT.at[fs, sid, :], osem)

    def wait_put():
        pltpu.make_async_copy(obuf, outT.at[0, sid, :], osem).wait()

    fetch(0, 0)
    for fi in range(FPC):
        slot = fi % 2
        wait_fetch(slot)
        if fi + 1 < FPC:
            fetch(fi + 1, 1 - slot)

        rowbuf = rows[slot]
        idxbuf = idxs[slot]

        if fi > 0:
            wait_put()

        def run_main():
            @plsc.parallel_loop(0, B // 16, unroll=UNROLL)
            def _body(j):
                idx16 = idxbuf[pl.ds(j * 16, 16)]
                obuf[pl.ds(j * 16, 16)] = plsc.load_gather(rowbuf, [idx16])

        def run_tail():
            @plsc.parallel_loop(0, B // 16, unroll=UNROLL)
            def _body(j):
                idx16 = idxbuf[pl.ds(j * 16, 16)]
                main = plsc.load_gather(rowbuf, [idx16])
                tval = plsc.load_gather(tailbuf, [jnp.maximum(idx16 - W, 0)])
                obuf[pl.ds(j * 16, 16)] = jnp.where(idx16 >= W, tval, main)

        if fi == FPC - 1:
            pl.when(cid < NUM_CORES - 1)(run_main)
            pl.when(cid == NUM_CORES - 1)(run_tail)
        else:
            run_main()

        put(fi)
    wait_put()


def _slab_starts():
    starts = [(f * PER_FIELD // 128) * 128 for f in range(F - 1)] + [A_LAST]
    return jnp.asarray(starts, dtype=jnp.int32)


def kernel(token_fields, table):
    tfT = token_fields.T
    tabT = table.T
    offs = jnp.arange(F, dtype=jnp.int32) * PER_FIELD - _slab_starts()
    local = tfT + offs[:, None]
    tail = jnp.pad(tabT[:, TAIL0:], ((0, 0), (0, NTAIL_PAD - NTAIL)))
    tail_lin = tail.reshape(D * NTAIL_PAD)
    outT = _fm_kernel(local, tabT, tail_lin)
    return jnp.transpose(outT, (2, 0, 1))
